# baseline (speedup 1.0000x reference)
import functools

import jax
import jax.numpy as jnp
from jax import lax
from jax.experimental import pallas as pl
from jax.experimental.pallas import tpu as pltpu
from jax.experimental.pallas import tpu_sc as plsc

_D = 128
_B = 4096
_LAMBDA = 0.0005

_info = plsc.get_sparse_core_info()
_NC, _NS, _L = _info.num_cores, _info.num_subcores, _info.num_lanes
_NW = _NC * _NS
_BPW = _B // _NW
_GPR = _D // _L
_OROWS = _BPW // 8
_OSLC = _OROWS // 4

_mesh = plsc.VectorSubcoreMesh(core_axis_name="c", subcore_axis_name="s")


@functools.partial(
    pl.kernel,
    mesh=_mesh,
    out_type=jax.ShapeDtypeStruct((_B // 8, _D), jnp.float32),
    scratch_types=[
        pltpu.VMEM((_BPW,), jnp.int32),
        pltpu.VMEM((_BPW, _D), jnp.float32),
        pltpu.VMEM((_BPW, _D), jnp.float32),
        pltpu.VMEM((_OROWS, _D), jnp.float32),
        pltpu.SemaphoreType.DMA,
        pltpu.SemaphoreType.DMA,
        pltpu.SemaphoreType.DMA,
    ],
)
def _sc_partials(feat_hbm, labels_hbm, centers_hbm, out_hbm,
                 idx_v, rows_v, feat_v, out_v, sem_g, sem_f, sem_o):
    wid = lax.axis_index("s") * _NC + lax.axis_index("c")
    base = wid * _BPW
    pltpu.sync_copy(labels_hbm.at[pl.ds(base, _BPW)], idx_v)
    gather_cp = pltpu.async_copy(centers_hbm.at[idx_v], rows_v, sem_g)
    feat_cp = pltpu.async_copy(feat_hbm.at[pl.ds(base, _BPW)], feat_v, sem_f)
    gather_cp.wait()
    feat_cp.wait()

    def packed_body(j, carry):
        for r in range(8):
            i = j * 8 + r
            acc = jnp.zeros((_L,), jnp.float32)
            for d in range(_GPR):
                f = feat_v[i, pl.ds(d * _L, _L)]
                c = rows_v[i, pl.ds(d * _L, _L)]
                df = f - c
                acc = acc + df * df
            out_v[j, pl.ds(r * _L, _L)] = acc
        return carry

    out_cps = []
    for q in range(4):
        lax.fori_loop(q * _OSLC, (q + 1) * _OSLC, packed_body, 0)
        out_cps.append(pltpu.async_copy(
            out_v.at[pl.ds(q * _OSLC, _OSLC)],
            out_hbm.at[pl.ds(wid * _OROWS + q * _OSLC, _OSLC)],
            sem_o))
    for cp in out_cps:
        cp.wait()


def _tc_finish_body(partials_ref, out_ref):
    x = partials_ref[...]
    cols = lax.broadcasted_iota(jnp.int32, (_D, 8), 0)
    groups = lax.broadcasted_iota(jnp.int32, (_D, 8), 1)
    g = (cols // _L == groups).astype(jnp.float32)
    sumsq = jnp.dot(x, g, preferred_element_type=jnp.float32)
    out_ref[0, 0] = jnp.sum(jnp.sqrt(sumsq)) * (_LAMBDA / _B)


@jax.jit
def _impl(features, labels, centers):
    partials = _sc_partials(features, labels.astype(jnp.int32), centers)
    loss = pl.pallas_call(
        _tc_finish_body,
        out_shape=jax.ShapeDtypeStruct((1, 1), jnp.float32),
        out_specs=pl.BlockSpec(memory_space=pltpu.SMEM),
    )(partials)
    return loss.reshape(())


def kernel(features, labels, centers):
    return _impl(features, labels, centers)


# speedup vs baseline: 1.0599x; 1.0599x over previous
import functools

import jax
import jax.numpy as jnp
from jax import lax
from jax.experimental import pallas as pl
from jax.experimental.pallas import tpu as pltpu
from jax.experimental.pallas import tpu_sc as plsc

_D = 128
_B = 4096
_LAMBDA = 0.0005

_info = plsc.get_sparse_core_info()
_NC, _NS, _L = _info.num_cores, _info.num_subcores, _info.num_lanes
_NW = _NC * _NS
_BPW = _B // _NW
_GPR = _D // _L
_OROWS = _BPW // 8

_mesh = plsc.VectorSubcoreMesh(core_axis_name="c", subcore_axis_name="s")


@functools.partial(
    pl.kernel,
    mesh=_mesh,
    out_type=jax.ShapeDtypeStruct((_B // 8, _D), jnp.float32),
    scratch_types=[
        pltpu.VMEM((_BPW,), jnp.int32),
        pltpu.VMEM((_BPW, _D), jnp.float32),
        pltpu.VMEM((_BPW, _D), jnp.float32),
        pltpu.VMEM((_OROWS, _D), jnp.float32),
        pltpu.SemaphoreType.DMA,
        pltpu.SemaphoreType.DMA,
    ],
)
def _sc_partials(feat_hbm, labels_hbm, centers_hbm, out_hbm,
                 idx_v, rows_v, feat_v, out_v, sem_g, sem_f):
    wid = lax.axis_index("s") * _NC + lax.axis_index("c")
    base = wid * _BPW
    pltpu.sync_copy(labels_hbm.at[pl.ds(base, _BPW)], idx_v)
    gather_cp = pltpu.async_copy(centers_hbm.at[idx_v], rows_v, sem_g)
    feat_cp = pltpu.async_copy(feat_hbm.at[pl.ds(base, _BPW)], feat_v, sem_f)
    gather_cp.wait()
    feat_cp.wait()

    def row_body(i, carry):
        acc = jnp.zeros((_L,), jnp.float32)
        for d in range(_GPR):
            f = feat_v[i, pl.ds(d * _L, _L)]
            c = rows_v[i, pl.ds(d * _L, _L)]
            df = f - c
            acc = acc + df * df
        out_v[i // 8, pl.ds((i % 8) * _L, _L)] = acc
        return carry

    lax.fori_loop(0, _BPW, row_body, 0)
    pltpu.sync_copy(out_v, out_hbm.at[pl.ds(wid * _OROWS, _OROWS)])


def _tc_finish_body(partials_ref, out_ref):
    x = partials_ref[...]
    cols = lax.broadcasted_iota(jnp.int32, (_D, 8), 0)
    groups = lax.broadcasted_iota(jnp.int32, (_D, 8), 1)
    g = (cols // _L == groups).astype(jnp.float32)
    sumsq = jnp.dot(x, g, preferred_element_type=jnp.float32)
    out_ref[0, 0] = jnp.sum(jnp.sqrt(sumsq)) * (_LAMBDA / _B)


@jax.jit
def _impl(features, labels, centers):
    partials = _sc_partials(features, labels.astype(jnp.int32), centers)
    loss = pl.pallas_call(
        _tc_finish_body,
        out_shape=jax.ShapeDtypeStruct((1, 1), jnp.float32),
        out_specs=pl.BlockSpec(memory_space=pltpu.SMEM),
    )(partials)
    return loss.reshape(())


def kernel(features, labels, centers):
    return _impl(features, labels, centers)
